# baseline (speedup 1.0000x reference)
import functools

import jax
import jax.numpy as jnp
from jax import lax
from jax.experimental import pallas as pl
from jax.experimental.pallas import tpu as pltpu
from jax.experimental.pallas import tpu_sc as plsc

_VOCAB = 1000000
_REPR = 64
_BATCH = 16384

_NC = 2
_NS = 16
_L = 16
_NW = _NC * _NS
_BPW = _BATCH // _NW
_GCH = 128
_NG = _BPW // _GCH
_W2 = 2 * _REPR

_mesh = plsc.VectorSubcoreMesh(core_axis_name="c", subcore_axis_name="s")


@functools.partial(
    pl.kernel,
    mesh=_mesh,
    compiler_params=pltpu.CompilerParams(use_tc_tiling_on_sc=True,
                                         needs_layout_passes=False),
    out_type=jax.ShapeDtypeStruct((_NW * _L,), jnp.float32),
    scratch_types=[
        pltpu.VMEM((_BPW,), jnp.int32),
        pltpu.VMEM((_BPW,), jnp.int32),
        pltpu.VMEM((_BPW,), jnp.int32),
        pltpu.VMEM((_BPW,), jnp.int32),
        pltpu.VMEM((_BPW,), jnp.int32),
        pltpu.VMEM((_BPW,), jnp.int32),
        pltpu.VMEM((2, _GCH, _W2), jnp.float32),
        pltpu.VMEM((2, _GCH, _W2), jnp.float32),
        pltpu.VMEM((2, _GCH // 2, _W2), jnp.float32),
        pltpu.VMEM((_L,), jnp.float32),
        pltpu.SemaphoreType.DMA,
        pltpu.SemaphoreType.DMA,
    ],
)
def _mse_partials(rep_hbm, ia_hbm, ib_hbm, emb_hbm, out_hbm,
                  ia_v, ib_v, ra_v, rb_v, oa_v, ob_v,
                  ea_v, eb_v, rep_v, acc_v, sem0, sem1):
    wid = lax.axis_index("s") * _NC + lax.axis_index("c")
    base = wid * _BPW
    rbase = wid * (_BPW // 2)

    pltpu.sync_copy(ia_hbm.at[pl.ds(base, _BPW)], ia_v)
    pltpu.sync_copy(ib_hbm.at[pl.ds(base, _BPW)], ib_v)
    for u in range(_BPW // _L):
        sl = pl.ds(u * _L, _L)
        va = ia_v[sl]
        vb = ib_v[sl]
        ra_v[sl] = va >> 1
        oa_v[sl] = (va & 1) << 6
        rb_v[sl] = vb >> 1
        ob_v[sl] = (vb & 1) << 6

    sems = (sem0, sem1)

    def fire(g):
        s = g % 2
        isl = pl.ds(g * _GCH, _GCH)
        return (
            pltpu.async_copy(emb_hbm.at[ra_v.at[isl]], ea_v.at[s], sems[s]),
            pltpu.async_copy(emb_hbm.at[rb_v.at[isl]], eb_v.at[s], sems[s]),
            pltpu.async_copy(
                rep_hbm.at[pl.ds(rbase + g * (_GCH // 2), _GCH // 2)],
                rep_v.at[s], sems[s]),
        )

    acc = jnp.zeros((_L,), jnp.float32)
    pending = {0: fire(0)}
    for g in range(_NG):
        if g + 1 < _NG:
            pending[g + 1] = fire(g + 1)
        for c in pending.pop(g):
            c.wait()
        s = g % 2
        for u in range(_GCH // _L):
            lrel = u * _L + lax.iota(jnp.int32, _L)
            lsl = pl.ds(g * _GCH + u * _L, _L)
            oa16 = oa_v[lsl]
            ob16 = ob_v[lsl]
            rrow16 = lrel >> 1
            rcol16 = (lrel & 1) << 6

            def body(c, acc, lrel=lrel, oa16=oa16, ob16=ob16,
                     rrow16=rrow16, rcol16=rcol16, s=s):
                va = plsc.load_gather(ea_v.at[s], [lrel, oa16 + c])
                vb = plsc.load_gather(eb_v.at[s], [lrel, ob16 + c])
                vr = plsc.load_gather(rep_v.at[s], [rrow16, rcol16 + c])
                d = va + vb - vr
                return acc + d * d

            acc = lax.fori_loop(0, _REPR, body, acc, unroll=8)

    acc_v[...] = acc
    pltpu.sync_copy(acc_v, out_hbm.at[pl.ds(wid * _L, _L)])


def kernel(rep, idx_a, idx_b, emb):
    emb2 = jnp.reshape(emb, (_VOCAB // 2, _W2))
    rep2 = jnp.reshape(rep, (_BATCH // 2, _W2))
    partials = _mse_partials(rep2, idx_a.astype(jnp.int32),
                             idx_b.astype(jnp.int32), emb2)
    return jnp.sum(partials) / jnp.float32(_BATCH * _REPR)


# speedup vs baseline: 1.1086x; 1.1086x over previous
import functools

import jax
import jax.numpy as jnp
from jax import lax
from jax.experimental import pallas as pl
from jax.experimental.pallas import tpu as pltpu
from jax.experimental.pallas import tpu_sc as plsc

_VOCAB = 1000000
_REPR = 64
_BATCH = 16384

_NC = 2
_NS = 16
_L = 16
_NW = _NC * _NS
_BPW = _BATCH // _NW
_GCH = 128
_NG = _BPW // _GCH

_mesh = plsc.VectorSubcoreMesh(core_axis_name="c", subcore_axis_name="s")


@functools.partial(
    pl.kernel,
    mesh=_mesh,
    compiler_params=pltpu.CompilerParams(use_tc_tiling_on_sc=True,
                                         needs_layout_passes=False),
    out_type=jax.ShapeDtypeStruct((_NW * _L,), jnp.float32),
    scratch_types=[
        pltpu.VMEM((_BPW,), jnp.int32),
        pltpu.VMEM((_BPW,), jnp.int32),
        pltpu.VMEM((2, _GCH, 128), jnp.float32),
        pltpu.VMEM((2, _GCH, 128), jnp.float32),
        pltpu.VMEM((2, _GCH // 2, 128), jnp.float32),
        pltpu.VMEM((_L,), jnp.float32),
        pltpu.SemaphoreType.DMA,
        pltpu.SemaphoreType.DMA,
    ],
)
def _mse_partials(rep_hbm, ia_hbm, ib_hbm, emb_hbm, out_hbm,
                  ia_v, ib_v, ea_v, eb_v, rep_v, acc_v, sem0, sem1):
    wid = lax.axis_index("s") * _NC + lax.axis_index("c")
    base = wid * _BPW
    rbase = wid * (_BPW // 2)

    pltpu.sync_copy(ia_hbm.at[pl.ds(base, _BPW)], ia_v)
    pltpu.sync_copy(ib_hbm.at[pl.ds(base, _BPW)], ib_v)

    sems = (sem0, sem1)

    def fire(g):
        s = g % 2
        isl = pl.ds(g * _GCH, _GCH)
        return (
            pltpu.async_copy(emb_hbm.at[ia_v.at[isl]], ea_v.at[s], sems[s]),
            pltpu.async_copy(emb_hbm.at[ib_v.at[isl]], eb_v.at[s], sems[s]),
            pltpu.async_copy(
                rep_hbm.at[pl.ds(rbase + g * (_GCH // 2), _GCH // 2)],
                rep_v.at[s], sems[s]),
        )

    acc = jnp.zeros((_L,), jnp.float32)
    lane = lax.iota(jnp.int32, _L)
    pending = {0: fire(0)}
    for g in range(_NG):
        if g + 1 < _NG:
            pending[g + 1] = fire(g + 1)
        for c in pending.pop(g):
            c.wait()
        s = g % 2
        for u in range(_GCH // _L):
            lrel = u * _L + lane
            rrow16 = lrel >> 1
            rcol16 = (lrel & 1) << 6

            def body(c, acc, lrel=lrel, rrow16=rrow16, rcol16=rcol16, s=s):
                c16 = jnp.zeros((_L,), jnp.int32) + c
                va = plsc.load_gather(ea_v.at[s], [lrel, c16])
                vb = plsc.load_gather(eb_v.at[s], [lrel, c16])
                vr = plsc.load_gather(rep_v.at[s], [rrow16, rcol16 + c])
                d = va + vb - vr
                return acc + d * d

            acc = lax.fori_loop(0, _REPR, body, acc, unroll=8)

    acc_v[...] = acc
    pltpu.sync_copy(acc_v, out_hbm.at[pl.ds(wid * _L, _L)])


def kernel(rep, idx_a, idx_b, emb):
    embp = jnp.pad(emb, ((0, 0), (0, 128 - _REPR)))
    rep2 = jnp.reshape(rep, (_BATCH // 2, 128))
    partials = _mse_partials(rep2, idx_a.astype(jnp.int32),
                             idx_b.astype(jnp.int32), embp)
    return jnp.sum(partials) / jnp.float32(_BATCH * _REPR)


# speedup vs baseline: 2.5410x; 2.2922x over previous
import functools

import jax
import jax.numpy as jnp
from jax import lax
from jax.experimental import pallas as pl
from jax.experimental.pallas import tpu as pltpu
from jax.experimental.pallas import tpu_sc as plsc

_VOCAB = 1000000
_REPR = 64
_BATCH = 16384

_NC = 2
_NS = 16
_L = 16
_NW = _NC * _NS
_BPW = _BATCH // _NW

_NBLK = _VOCAB // 128
_TAIL0 = _NBLK * 128
_WBLK = 244
_WSPAN = 248
_CPW = _WSPAN // 4
_CAP = 3072

_mesh = plsc.VectorSubcoreMesh(core_axis_name="c", subcore_axis_name="s")
_IOTA = None


@functools.partial(
    pl.kernel,
    mesh=_mesh,
    compiler_params=pltpu.CompilerParams(use_tc_tiling_on_sc=True,
                                         needs_layout_passes=False),
    out_type=(jax.ShapeDtypeStruct((_BATCH * _REPR,), jnp.float32),
              jax.ShapeDtypeStruct((_BATCH * _REPR,), jnp.float32)),
    scratch_types=[
        pltpu.VMEM((_BATCH,), jnp.int32),
        pltpu.VMEM((_BATCH,), jnp.int32),
        pltpu.VMEM((2, 256, 128), jnp.float32),
        pltpu.VMEM((_CAP + 16,), jnp.int32),
        pltpu.VMEM((_CAP + 16,), jnp.int32),
        pltpu.VMEM((_CAP + 16,), jnp.int32),
        pltpu.VMEM((_CAP + 16,), jnp.int32),
        pltpu.VMEM((16 * _REPR,), jnp.float32),
        pltpu.SemaphoreType.DMA,
        pltpu.SemaphoreType.DMA,
        pltpu.SemaphoreType.DMA,
    ],
)
def _extract(ia_hbm, ib_hbm, embt_hbm, ea_hbm, eb_hbm,
             ia_v, ib_v, sbuf_v, mi_v, ml_v, mi2_v, ml2_v, stage_v,
             sem0, sem1, semo):
    wid = lax.axis_index("s") * _NC + lax.axis_index("c")
    lane = lax.iota(jnp.int32, _L)
    b0 = wid * _WBLK
    lo = b0 * 128
    hi = jnp.minimum((b0 + _WSPAN) * 128, _TAIL0)

    pltpu.sync_copy(ia_hbm, ia_v)
    pltpu.sync_copy(ib_hbm, ib_v)

    inval = jnp.zeros((_L,), jnp.int32) + (1 << 30)
    for t in range((_CAP + 16) // _L):
        mi_v[pl.ds(t * _L, _L)] = inval
        mi2_v[pl.ds(t * _L, _L)] = inval

    def scan_set(idx_ref, slot_base, r0):
        def body(u, r_base):
            v = idx_ref[pl.ds(u * _L, _L)]
            m = (v >= lo) & (v < hi)
            mi = m.astype(jnp.int32)
            rank = r_base + plsc.cumsum(mi) - mi
            acc_m = m & (rank < _CAP)
            pos = jnp.minimum(rank, _CAP + 15)
            plsc.store_scatter(mi_v, [pos], v, mask=acc_m)
            plsc.store_scatter(ml_v, [pos], slot_base + u * _L + lane,
                               mask=acc_m)
            return r_base + jnp.sum(mi)
        return lax.fori_loop(0, _BATCH // _L, body, r0)

    nmatch = scan_set(ib_v, _BATCH, scan_set(ia_v, 0, jnp.int32(0)))
    cnt = jnp.minimum(nmatch, _CAP)
    ngrp = (cnt + _L - 1) // _L
    overflow = nmatch > _CAP

    sems = (sem0, sem1)

    def chunk_start(j):
        return jnp.minimum(b0 + 4 * j, _NBLK - 4) * 128

    def fire(j, s):
        cs = chunk_start(j)
        for q in range(4):
            pltpu.async_copy(embt_hbm.at[:, pl.ds(cs + q * 128, 128)],
                             sbuf_v.at[s, pl.ds(q * 64, 64)], sems[s])

    def drain(s):
        for q in range(4):
            pltpu.make_async_copy(embt_hbm.at[:, pl.ds(0, 128)],
                                  sbuf_v.at[s, pl.ds(q * 64, 64)],
                                  sems[s]).wait()

    def extract_one(s, cs, idx, slot, e):
        rel = idx - cs
        rbase = (rel >> 7) * 64
        col = jnp.zeros((_L,), jnp.int32) + (rel & 127)
        ring = (e & 15) * _REPR
        for qf in range(4):
            row16 = rbase + qf * _L + lane
            va = plsc.load_gather(sbuf_v.at[s], [row16, col])
            stage_v[pl.ds(ring + qf * _L, _L)] = va
        is_b = slot >= _BATCH
        lsel = slot & (_BATCH - 1)
        src = stage_v.at[pl.ds(ring, _REPR)]

        @pl.when(is_b)
        def _():
            pltpu.async_copy(src, eb_hbm.at[pl.ds(lsel * _REPR, _REPR)], semo)

        @pl.when(jnp.logical_not(is_b))
        def _():
            pltpu.async_copy(src, ea_hbm.at[pl.ds(lsel * _REPR, _REPR)], semo)

    def drain_out(n):
        def body(t, _):
            pltpu.make_async_copy(ea_hbm.at[pl.ds(0, _REPR)],
                                  stage_v.at[pl.ds(0, _REPR)], semo).wait()
            return 0
        lax.fori_loop(0, n, body, 0)

    def process(j, s):
        drain(s)
        cs = chunk_start(j)

        def bin_body(u, c2):
            mi16 = mi_v[pl.ds(u * _L, _L)]
            m = (mi16 >= cs) & (mi16 < cs + 512)
            mi_i = m.astype(jnp.int32)
            rank = c2 + plsc.cumsum(mi_i) - mi_i
            pos = jnp.minimum(rank, _CAP + 15)
            plsc.store_scatter(mi2_v, [pos], mi16, mask=m)
            ml16 = ml_v[pl.ds(u * _L, _L)]
            plsc.store_scatter(ml2_v, [pos], ml16, mask=m)
            return c2 + jnp.sum(mi_i)

        ccnt = lax.fori_loop(0, ngrp, bin_body, jnp.int32(0))

        def ext_grp(u, _):
            mi16 = mi2_v[pl.ds(u * _L, _L)]
            ml16 = ml2_v[pl.ds(u * _L, _L)]
            nrem = ccnt - u * _L
            for t in range(_L):
                @pl.when(t < nrem)
                def _(t=t):
                    extract_one(s, cs, mi16[t], ml16[t], t)
            drain_out(jnp.minimum(nrem, _L))
            return 0

        lax.fori_loop(0, (ccnt + _L - 1) // _L, ext_grp, 0)

        @pl.when(overflow)
        def _():
            def rescan(idx_ref, slot_base):
                def body(u, e):
                    v = idx_ref[pl.ds(u * _L, _L)]
                    m = ((v >= cs) & (v < cs + 512)).astype(jnp.int32)
                    nm = jnp.sum(m)

                    def slow(e):
                        for t in range(_L):
                            @pl.when(m[t] > 0)
                            def _(t=t):
                                extract_one(s, cs, v[t],
                                            slot_base + u * _L + t, t)
                        drain_out(nm)
                        return e + _L

                    return lax.cond(nm > 0, slow, lambda e: e, e)
                return lax.fori_loop(0, _BATCH // _L, body, jnp.int32(0))

            rescan(ia_v, 0)
            rescan(ib_v, _BATCH)

    fire(jnp.int32(0), 0)

    def pair_body(p, _):
        j0 = 2 * p
        fire(j0 + 1, 1)
        process(j0, 0)

        @pl.when(p < _CPW // 2 - 1)
        def _():
            fire(j0 + 2, 0)

        process(j0 + 1, 1)
        return 0

    lax.fori_loop(0, _CPW // 2, pair_body, 0)


@functools.partial(
    pl.kernel,
    mesh=_mesh,
    compiler_params=pltpu.CompilerParams(use_tc_tiling_on_sc=True,
                                         needs_layout_passes=False),
    out_type=jax.ShapeDtypeStruct((_NW * _L,), jnp.float32),
    scratch_types=[
        pltpu.VMEM((_BPW,), jnp.int32),
        pltpu.VMEM((_BPW,), jnp.int32),
        pltpu.VMEM((_BPW * _REPR,), jnp.float32),
        pltpu.VMEM((_BPW * _REPR,), jnp.float32),
        pltpu.VMEM((4, 64, 128), jnp.float32),
        pltpu.VMEM((64, 128), jnp.float32),
        pltpu.VMEM((_L,), jnp.float32),
        pltpu.SemaphoreType.DMA,
    ],
)
def _mse(ia_hbm, ib_hbm, rept_hbm, tail_hbm, ea_hbm, eb_hbm, out_hbm,
         ia_v, ib_v, ea_v, eb_v, rep_v, tail_v, acc_v, sem):
    wid = lax.axis_index("s") * _NC + lax.axis_index("c")
    lane = lax.iota(jnp.int32, _L)
    base = wid * _BPW

    cps = [
        pltpu.async_copy(ia_hbm.at[pl.ds(base, _BPW)], ia_v, sem),
        pltpu.async_copy(ib_hbm.at[pl.ds(base, _BPW)], ib_v, sem),
        pltpu.async_copy(ea_hbm.at[pl.ds(base * _REPR, _BPW * _REPR)],
                         ea_v, sem),
        pltpu.async_copy(eb_hbm.at[pl.ds(base * _REPR, _BPW * _REPR)],
                         eb_v, sem),
        pltpu.async_copy(tail_hbm, tail_v, sem),
    ]
    for q in range(4):
        cps.append(pltpu.async_copy(
            rept_hbm.at[:, pl.ds(base + q * 128, 128)], rep_v.at[q], sem))
    for c in cps:
        c.wait()

    acc = jnp.zeros((_L,), jnp.float32)
    for u in range(_BPW // _L):
        l16 = u * _L + lane
        fbase = l16 * _REPR
        va_i = ia_v[pl.ds(u * _L, _L)]
        vb_i = ib_v[pl.ds(u * _L, _L)]
        ma = va_i >= _TAIL0
        mb = vb_i >= _TAIL0
        ca = jnp.maximum(va_i - _TAIL0, 0)
        cb = jnp.maximum(vb_i - _TAIL0, 0)
        colr = l16 & 127
        q = (u * _L) // 128

        def body(c, acc, fbase=fbase, ma=ma, mb=mb, ca=ca, cb=cb,
                 colr=colr, q=q):
            c16 = jnp.zeros((_L,), jnp.int32) + c
            va = plsc.load_gather(ea_v, [fbase + c])
            vb = plsc.load_gather(eb_v, [fbase + c])
            va = jnp.where(ma, plsc.load_gather(tail_v, [c16, ca]), va)
            vb = jnp.where(mb, plsc.load_gather(tail_v, [c16, cb]), vb)
            vr = plsc.load_gather(rep_v.at[q], [c16, colr])
            d = va + vb - vr
            return acc + d * d

        acc = lax.fori_loop(0, _REPR, body, acc, unroll=4)

    acc_v[...] = acc
    pltpu.sync_copy(acc_v, out_hbm.at[pl.ds(wid * _L, _L)])


def kernel(rep, idx_a, idx_b, emb):
    ia = idx_a.astype(jnp.int32)
    ib = idx_b.astype(jnp.int32)
    embt = emb.T
    rept = rep.T
    tail = jnp.pad(embt[:, _TAIL0:], ((0, 0), (0, 64)))
    ea, eb = _extract(ia, ib, embt)
    partials = _mse(ia, ib, rept, tail, ea, eb)
    return jnp.sum(partials) / jnp.float32(_BATCH * _REPR)


# speedup vs baseline: 2.5914x; 1.0198x over previous
import functools

import jax
import jax.numpy as jnp
from jax import lax
from jax.experimental import pallas as pl
from jax.experimental.pallas import tpu as pltpu
from jax.experimental.pallas import tpu_sc as plsc

_VOCAB = 1000000
_REPR = 64
_BATCH = 16384

_NC = 2
_NS = 16
_L = 16
_NW = _NC * _NS
_BPW = _BATCH // _NW

_NBLK = _VOCAB // 128
_TAIL0 = _NBLK * 128
_WBLK = 244
_WSPAN = 248
_CPW = _WSPAN // 4
_CAP = 3072

_mesh = plsc.VectorSubcoreMesh(core_axis_name="c", subcore_axis_name="s")
_IOTA = None


@functools.partial(
    pl.kernel,
    mesh=_mesh,
    compiler_params=pltpu.CompilerParams(use_tc_tiling_on_sc=True,
                                         needs_layout_passes=False),
    out_type=(jax.ShapeDtypeStruct((_BATCH * _REPR,), jnp.float32),
              jax.ShapeDtypeStruct((_BATCH * _REPR,), jnp.float32)),
    scratch_types=[
        pltpu.VMEM((_BATCH,), jnp.int32),
        pltpu.VMEM((_BATCH,), jnp.int32),
        pltpu.VMEM((2, 256, 128), jnp.float32),
        pltpu.VMEM((_CAP + 16,), jnp.int32),
        pltpu.VMEM((_CAP + 16,), jnp.int32),
        pltpu.VMEM((_CAP + 16,), jnp.int32),
        pltpu.VMEM((_CAP + 16,), jnp.int32),
        pltpu.VMEM((16 * _REPR,), jnp.float32),
        pltpu.SemaphoreType.DMA,
        pltpu.SemaphoreType.DMA,
        pltpu.SemaphoreType.DMA,
    ],
)
def _extract(ia_hbm, ib_hbm, embt_hbm, ea_hbm, eb_hbm,
             ia_v, ib_v, sbuf_v, mi_v, ml_v, mi2_v, ml2_v, stage_v,
             sem0, sem1, semo):
    wid = lax.axis_index("s") * _NC + lax.axis_index("c")
    lane = lax.iota(jnp.int32, _L)
    b0 = wid * _WBLK
    lo = b0 * 128
    hi = jnp.minimum((b0 + _WSPAN) * 128, _TAIL0)

    pltpu.sync_copy(ia_hbm, ia_v)
    pltpu.sync_copy(ib_hbm, ib_v)

    inval = jnp.zeros((_L,), jnp.int32) + (1 << 30)
    for t in range((_CAP + 16) // _L):
        mi_v[pl.ds(t * _L, _L)] = inval
        mi2_v[pl.ds(t * _L, _L)] = inval

    def scan_set(idx_ref, slot_base, r0):
        def body(u, r_base):
            v = idx_ref[pl.ds(u * _L, _L)]
            m = (v >= lo) & (v < hi)
            mi = m.astype(jnp.int32)
            rank = r_base + plsc.cumsum(mi) - mi
            acc_m = m & (rank < _CAP)
            pos = jnp.minimum(rank, _CAP + 15)
            plsc.store_scatter(mi_v, [pos], v, mask=acc_m)
            plsc.store_scatter(ml_v, [pos], slot_base + u * _L + lane,
                               mask=acc_m)
            return r_base + jnp.sum(mi)
        return lax.fori_loop(0, _BATCH // _L, body, r0)

    nmatch = scan_set(ib_v, _BATCH, scan_set(ia_v, 0, jnp.int32(0)))
    cnt = jnp.minimum(nmatch, _CAP)
    ngrp = (cnt + _L - 1) // _L
    overflow = nmatch > _CAP

    sems = (sem0, sem1)

    def chunk_start(j):
        return jnp.minimum(b0 + 4 * j, _NBLK - 4) * 128

    def fire(j, s):
        cs = chunk_start(j)
        for q in range(4):
            pltpu.async_copy(embt_hbm.at[:, pl.ds(cs + q * 128, 128)],
                             sbuf_v.at[s, pl.ds(q * 64, 64)], sems[s])

    def drain(s):
        for q in range(4):
            pltpu.make_async_copy(embt_hbm.at[:, pl.ds(0, 128)],
                                  sbuf_v.at[s, pl.ds(q * 64, 64)],
                                  sems[s]).wait()

    def extract_one(s, cs, idx, slot, e):
        rel = idx - cs
        rbase = (rel >> 7) * 64
        col = jnp.zeros((_L,), jnp.int32) + (rel & 127)
        ring = (e & 15) * _REPR
        for qf in range(4):
            row16 = rbase + qf * _L + lane
            va = plsc.load_gather(sbuf_v.at[s], [row16, col])
            stage_v[pl.ds(ring + qf * _L, _L)] = va
        is_b = slot >= _BATCH
        lsel = slot & (_BATCH - 1)
        src = stage_v.at[pl.ds(ring, _REPR)]

        @pl.when(is_b)
        def _():
            pltpu.async_copy(src, eb_hbm.at[pl.ds(lsel * _REPR, _REPR)], semo)

        @pl.when(jnp.logical_not(is_b))
        def _():
            pltpu.async_copy(src, ea_hbm.at[pl.ds(lsel * _REPR, _REPR)], semo)

    def drain_out(n):
        def body(t, _):
            pltpu.make_async_copy(ea_hbm.at[pl.ds(0, _REPR)],
                                  stage_v.at[pl.ds(0, _REPR)], semo).wait()
            return 0
        lax.fori_loop(0, n, body, 0)

    def process(j, s):
        drain(s)
        cs = chunk_start(j)

        def bin_body(u, c2):
            mi16 = mi_v[pl.ds(u * _L, _L)]
            m = (mi16 >= cs) & (mi16 < cs + 512)
            mi_i = m.astype(jnp.int32)
            rank = c2 + plsc.cumsum(mi_i) - mi_i
            pos = jnp.minimum(rank, _CAP + 15)
            plsc.store_scatter(mi2_v, [pos], mi16, mask=m)
            ml16 = ml_v[pl.ds(u * _L, _L)]
            plsc.store_scatter(ml2_v, [pos], ml16, mask=m)
            return c2 + jnp.sum(mi_i)

        ccnt = lax.fori_loop(0, ngrp, bin_body, jnp.int32(0))

        def ext_grp(u, _):
            mi16 = mi2_v[pl.ds(u * _L, _L)]
            ml16 = ml2_v[pl.ds(u * _L, _L)]
            nrem = ccnt - u * _L
            for t in range(_L):
                @pl.when(t < nrem)
                def _(t=t):
                    extract_one(s, cs, mi16[t], ml16[t], t)
            drain_out(jnp.minimum(nrem, _L))
            return 0

        lax.fori_loop(0, (ccnt + _L - 1) // _L, ext_grp, 0)

        @pl.when(overflow)
        def _():
            def rescan(idx_ref, slot_base):
                def body(u, e):
                    v = idx_ref[pl.ds(u * _L, _L)]
                    m = ((v >= cs) & (v < cs + 512)).astype(jnp.int32)
                    nm = jnp.sum(m)

                    def slow(e):
                        for t in range(_L):
                            @pl.when(m[t] > 0)
                            def _(t=t):
                                extract_one(s, cs, v[t],
                                            slot_base + u * _L + t, t)
                        drain_out(nm)
                        return e + _L

                    return lax.cond(nm > 0, slow, lambda e: e, e)
                return lax.fori_loop(0, _BATCH // _L, body, jnp.int32(0))

            rescan(ia_v, 0)
            rescan(ib_v, _BATCH)

    fire(jnp.int32(0), 0)

    def pair_body(p, _):
        j0 = 2 * p
        fire(j0 + 1, 1)
        process(j0, 0)

        @pl.when(p < _CPW // 2 - 1)
        def _():
            fire(j0 + 2, 0)

        process(j0 + 1, 1)
        return 0

    lax.fori_loop(0, _CPW // 2, pair_body, 0)


@functools.partial(
    pl.kernel,
    mesh=_mesh,
    compiler_params=pltpu.CompilerParams(use_tc_tiling_on_sc=True,
                                         needs_layout_passes=False),
    out_type=jax.ShapeDtypeStruct((_NW * _L,), jnp.float32),
    scratch_types=[
        pltpu.VMEM((_BPW,), jnp.int32),
        pltpu.VMEM((_BPW,), jnp.int32),
        pltpu.VMEM((_BPW * _REPR,), jnp.float32),
        pltpu.VMEM((_BPW * _REPR,), jnp.float32),
        pltpu.VMEM((4, 64, 128), jnp.float32),
        pltpu.VMEM((64, 128), jnp.float32),
        pltpu.VMEM((_L,), jnp.float32),
        pltpu.SemaphoreType.DMA,
    ],
)
def _mse(ia_hbm, ib_hbm, rept_hbm, tail_hbm, ea_hbm, eb_hbm, out_hbm,
         ia_v, ib_v, ea_v, eb_v, rep_v, tail_v, acc_v, sem):
    wid = lax.axis_index("s") * _NC + lax.axis_index("c")
    lane = lax.iota(jnp.int32, _L)
    base = wid * _BPW

    cps = [
        pltpu.async_copy(ia_hbm.at[pl.ds(base, _BPW)], ia_v, sem),
        pltpu.async_copy(ib_hbm.at[pl.ds(base, _BPW)], ib_v, sem),
        pltpu.async_copy(ea_hbm.at[pl.ds(base * _REPR, _BPW * _REPR)],
                         ea_v, sem),
        pltpu.async_copy(eb_hbm.at[pl.ds(base * _REPR, _BPW * _REPR)],
                         eb_v, sem),
        pltpu.async_copy(tail_hbm, tail_v, sem),
    ]
    for q in range(4):
        cps.append(pltpu.async_copy(
            rept_hbm.at[:, pl.ds(base + q * 128, 128)], rep_v.at[q], sem))
    for c in cps:
        c.wait()

    def patch_tail(idx_ref, dst_ref):
        def grp(u, _):
            v = idx_ref[pl.ds(u * _L, _L)]
            m = (v >= _TAIL0).astype(jnp.int32)

            @pl.when(jnp.sum(m) > 0)
            def _():
                for t in range(_L):
                    @pl.when(m[t] > 0)
                    def _(t=t):
                        col = jnp.zeros((_L,), jnp.int32) + (v[t] - _TAIL0)
                        for qf in range(4):
                            c16 = qf * _L + lane
                            tv = plsc.load_gather(tail_v, [c16, col])
                            dst_ref[pl.ds((u * _L + t) * _REPR + qf * _L,
                                          _L)] = tv
            return 0

        lax.fori_loop(0, _BPW // _L, grp, 0)

    patch_tail(ia_v, ea_v)
    patch_tail(ib_v, eb_v)

    acc = jnp.zeros((_L,), jnp.float32)
    for u in range(_BPW // _L):
        l16 = u * _L + lane
        fbase = l16 * _REPR
        colr = l16 & 127
        q = (u * _L) // 128

        def body(c, acc, fbase=fbase, colr=colr, q=q):
            c16 = jnp.zeros((_L,), jnp.int32) + c
            va = plsc.load_gather(ea_v, [fbase + c])
            vb = plsc.load_gather(eb_v, [fbase + c])
            vr = plsc.load_gather(rep_v.at[q], [c16, colr])
            d = va + vb - vr
            return acc + d * d

        acc = lax.fori_loop(0, _REPR, body, acc, unroll=8)

    acc_v[...] = acc
    pltpu.sync_copy(acc_v, out_hbm.at[pl.ds(wid * _L, _L)])


def kernel(rep, idx_a, idx_b, emb):
    ia = idx_a.astype(jnp.int32)
    ib = idx_b.astype(jnp.int32)
    embt = emb.T
    rept = rep.T
    tail = jnp.pad(embt[:, _TAIL0:], ((0, 0), (0, 64)))
    ea, eb = _extract(ia, ib, embt)
    partials = _mse(ia, ib, rept, tail, ea, eb)
    return jnp.sum(partials) / jnp.float32(_BATCH * _REPR)
